# baseline (speedup 1.0000x reference)
import functools

import jax
import jax.numpy as jnp
from jax import lax
from jax.experimental import pallas as pl
from jax.experimental.pallas import tpu as pltpu
from jax.experimental.pallas import tpu_sc as plsc

N = 320000
D = 128
NUM_TYPES = 8
EPS = 1e-05

BLK = 3200
NB = N // BLK

NC = 2
NS = 16
NW = NC * NS
M_SC = 179200
RW = M_SC // NW
CHUNK = 112
NCHUNK = RW // CHUNK
TV_REG = RW // 16
NB_TC = (N - M_SC) // BLK
TC_OFF = M_SC // BLK


def _sc_stats_body(x_hbm, tv_hbm, s1_hbm, s2_hbm, cnt_hbm,
                   tv_v, xbuf, s1_v, s2_v, cnt_v, sem0, sem1):
    wid = lax.axis_index("s") * NC + lax.axis_index("c")
    wstart = pl.multiple_of(wid * RW, 8)

    pltpu.sync_copy(tv_hbm.at[pl.ds(wstart, RW)], tv_v)

    def count_body(i, carry):
        tv = tv_v[pl.ds(i * 16, 16)]
        return tuple(carry[t - 1] + jnp.where(tv < t, 1, 0)
                     for t in range(1, NUM_TYPES))

    init = tuple(jnp.zeros((16,), jnp.int32) for _ in range(NUM_TYPES - 1))
    accs = lax.fori_loop(0, TV_REG, count_body, init)
    bounds = ([jnp.int32(0)] + [jnp.sum(a) for a in accs] + [jnp.int32(RW)])

    zero16 = jnp.zeros((16,), jnp.float32)
    for t in range(NUM_TYPES):
        for j in range(D // 16):
            s1_v[t, pl.ds(j * 16, 16)] = zero16
            s2_v[t, pl.ds(j * 16, 16)] = zero16

    def start_copy(k, buf, sem):
        off = pl.multiple_of(wstart + k * CHUNK, 8)
        pltpu.make_async_copy(
            x_hbm.at[pl.ds(off, CHUNK), :], buf, sem).start()

    def wait_copy(buf, sem):
        pltpu.make_async_copy(
            x_hbm.at[pl.ds(wstart, CHUNK), :], buf, sem).wait()

    def process(k, buf):
        base = k * CHUNK
        for t in range(NUM_TYPES):
            lo = jnp.clip(bounds[t] - base, 0, CHUNK)
            hi = jnp.clip(bounds[t + 1] - base, lo, CHUNK)

            def row_body(r, carry):
                c1, c2 = carry
                n1 = []
                n2 = []
                for j in range(D // 16):
                    v = buf[r, pl.ds(j * 16, 16)]
                    n1.append(c1[j] + v)
                    n2.append(c2[j] + v * v)
                return (tuple(n1), tuple(n2))

            rinit = (tuple(zero16 for _ in range(D // 16)),
                     tuple(zero16 for _ in range(D // 16)))
            r1, r2 = lax.fori_loop(lo, hi, row_body, rinit)

            @pl.when(hi > lo)
            def _flush():
                for j in range(D // 16):
                    s1_v[t, pl.ds(j * 16, 16)] += r1[j]
                    s2_v[t, pl.ds(j * 16, 16)] += r2[j]

    start_copy(0, xbuf.at[0], sem0)
    start_copy(1, xbuf.at[1], sem1)

    def pair_body(k, _):
        wait_copy(xbuf.at[0], sem0)
        process(2 * k, xbuf.at[0])

        @pl.when(k < NCHUNK // 2 - 1)
        def _next0():
            start_copy(2 * k + 2, xbuf.at[0], sem0)

        wait_copy(xbuf.at[1], sem1)
        process(2 * k + 1, xbuf.at[1])

        @pl.when(k < NCHUNK // 2 - 1)
        def _next1():
            start_copy(2 * k + 3, xbuf.at[1], sem1)
        return 0

    lax.fori_loop(0, NCHUNK // 2, pair_body, 0)

    for t in range(NUM_TYPES):
        c = (bounds[t + 1] - bounds[t]).astype(jnp.float32)
        cnt_v[t, :] = jnp.full((16,), c, jnp.float32)
    pltpu.sync_copy(s1_v, s1_hbm.at[wid])
    pltpu.sync_copy(s2_v, s2_hbm.at[wid])
    pltpu.sync_copy(cnt_v, cnt_hbm.at[wid])


_sc_stats = functools.partial(
    pl.kernel,
    out_type=[
        jax.ShapeDtypeStruct((NW, NUM_TYPES, D), jnp.float32),
        jax.ShapeDtypeStruct((NW, NUM_TYPES, D), jnp.float32),
        jax.ShapeDtypeStruct((NW, NUM_TYPES, 16), jnp.float32),
    ],
    mesh=plsc.VectorSubcoreMesh(core_axis_name="c", subcore_axis_name="s"),
    compiler_params=pltpu.CompilerParams(needs_layout_passes=False),
    scratch_types=[
        pltpu.VMEM((RW,), jnp.int32),
        pltpu.VMEM((2, CHUNK, D), jnp.float32),
        pltpu.VMEM((NUM_TYPES, D), jnp.float32),
        pltpu.VMEM((NUM_TYPES, D), jnp.float32),
        pltpu.VMEM((NUM_TYPES, 16), jnp.float32),
        pltpu.SemaphoreType.DMA,
        pltpu.SemaphoreType.DMA,
    ],
)(_sc_stats_body)


def _tc_stats_body(type_ref, x_ref, s1_out, s2_out, cnt_out,
                   s1_ref, s2_ref, cnt_ref):
    j = pl.program_id(0)

    @pl.when(j == 0)
    def _init():
        s1_ref[...] = jnp.zeros_like(s1_ref)
        s2_ref[...] = jnp.zeros_like(s2_ref)
        cnt_ref[...] = jnp.zeros_like(cnt_ref)

    t = type_ref[0, 0, :]
    xb = x_ref[...]
    oh = (jax.lax.broadcasted_iota(jnp.int32, (NUM_TYPES, BLK), 0)
          == t[None, :]).astype(jnp.float32)
    dn = (((1,), (0,)), ((), ()))
    s1_ref[...] += lax.dot_general(oh, xb, dn,
                                   preferred_element_type=jnp.float32)
    s2_ref[...] += lax.dot_general(oh, xb * xb, dn,
                                   preferred_element_type=jnp.float32)
    cnt_ref[...] += jnp.broadcast_to(jnp.sum(oh, axis=1)[:, None],
                                     (NUM_TYPES, D))

    @pl.when(j == NB_TC - 1)
    def _emit():
        s1_out[...] = s1_ref[...]
        s2_out[...] = s2_ref[...]
        cnt_out[...] = cnt_ref[...]


def _norm_body(type_ref, x_ref, s1_ref, s2_ref, cnt_ref,
               s1t_ref, s2t_ref, cntt_ref, out_ref,
               mean_ref, rstd_ref):
    j = pl.program_id(0)

    @pl.when(j == 0)
    def _finalize():
        c = jnp.sum(cnt_ref[...], axis=0)[:, :1] + cntt_ref[:, :1]
        c = jnp.maximum(c, 1.0)
        mean = (jnp.sum(s1_ref[...], axis=0) + s1t_ref[...]) / c
        var = (jnp.sum(s2_ref[...], axis=0) + s2t_ref[...]) / c - mean * mean
        var = jnp.maximum(var, 0.0)
        mean_ref[...] = mean
        rstd_ref[...] = lax.rsqrt(var + EPS)

    t = type_ref[0, 0, :]
    oh = (t[:, None] == lax.broadcasted_iota(
        jnp.int32, (BLK, NUM_TYPES), 1)).astype(jnp.float32)
    dn = (((1,), (0,)), ((), ()))
    m = lax.dot_general(oh, mean_ref[...], dn,
                        preferred_element_type=jnp.float32)
    r = lax.dot_general(oh, rstd_ref[...], dn,
                        preferred_element_type=jnp.float32)
    out_ref[...] = (x_ref[...] - m) * r


@jax.jit
def kernel(x, type_vec):
    tv = type_vec.astype(jnp.int32)
    tv3 = tv.reshape(NB, 1, BLK)

    s1p, s2p, cntp = _sc_stats(x, tv)

    s1t, s2t, cntt = pl.pallas_call(
        _tc_stats_body,
        grid=(NB_TC,),
        in_specs=[
            pl.BlockSpec((1, 1, BLK), lambda j: (TC_OFF + j, 0, 0)),
            pl.BlockSpec((BLK, D), lambda j: (TC_OFF + j, 0)),
        ],
        out_specs=[
            pl.BlockSpec((NUM_TYPES, D), lambda j: (0, 0)),
            pl.BlockSpec((NUM_TYPES, D), lambda j: (0, 0)),
            pl.BlockSpec((NUM_TYPES, D), lambda j: (0, 0)),
        ],
        out_shape=[
            jax.ShapeDtypeStruct((NUM_TYPES, D), jnp.float32),
            jax.ShapeDtypeStruct((NUM_TYPES, D), jnp.float32),
            jax.ShapeDtypeStruct((NUM_TYPES, D), jnp.float32),
        ],
        scratch_shapes=[
            pltpu.VMEM((NUM_TYPES, D), jnp.float32),
            pltpu.VMEM((NUM_TYPES, D), jnp.float32),
            pltpu.VMEM((NUM_TYPES, D), jnp.float32),
        ],
    )(tv3, x)

    out = pl.pallas_call(
        _norm_body,
        grid=(NB,),
        in_specs=[
            pl.BlockSpec((1, 1, BLK), lambda j: (j, 0, 0)),
            pl.BlockSpec((BLK, D), lambda j: (j, 0)),
            pl.BlockSpec((NW, NUM_TYPES, D), lambda j: (0, 0, 0)),
            pl.BlockSpec((NW, NUM_TYPES, D), lambda j: (0, 0, 0)),
            pl.BlockSpec((NW, NUM_TYPES, 16), lambda j: (0, 0, 0)),
            pl.BlockSpec((NUM_TYPES, D), lambda j: (0, 0)),
            pl.BlockSpec((NUM_TYPES, D), lambda j: (0, 0)),
            pl.BlockSpec((NUM_TYPES, D), lambda j: (0, 0)),
        ],
        out_specs=pl.BlockSpec((BLK, D), lambda j: (j, 0)),
        out_shape=jax.ShapeDtypeStruct((N, D), jnp.float32),
        scratch_shapes=[
            pltpu.VMEM((NUM_TYPES, D), jnp.float32),
            pltpu.VMEM((NUM_TYPES, D), jnp.float32),
        ],
    )(tv3, x, s1p, s2p, cntp, s1t, s2t, cntt)
    return out


# speedup vs baseline: 1.0286x; 1.0286x over previous
import functools

import jax
import jax.numpy as jnp
from jax import lax
from jax.experimental import pallas as pl
from jax.experimental.pallas import tpu as pltpu
from jax.experimental.pallas import tpu_sc as plsc

N = 320000
D = 128
NUM_TYPES = 8
EPS = 1e-05

BLK = 3200
NB = N // BLK

NC = 2
NS = 16
NW = NC * NS
M_SC = 179200
RW = M_SC // NW
CHUNK = 200
NCHUNK = RW // CHUNK
TV_REG = RW // 16
NB_TC = (N - M_SC) // BLK
TC_OFF = M_SC // BLK


def _sc_stats_body(x_hbm, tv_hbm, s1_hbm, s2_hbm, cnt_hbm,
                   tv_v, xbuf, s1_v, s2_v, cnt_v, sem0, sem1):
    wid = lax.axis_index("s") * NC + lax.axis_index("c")
    wstart = pl.multiple_of(wid * RW, 8)

    pltpu.sync_copy(tv_hbm.at[pl.ds(wstart, RW)], tv_v)

    def count_body(i, carry):
        tv = tv_v[pl.ds(i * 16, 16)]
        return tuple(carry[t - 1] + jnp.where(tv < t, 1, 0)
                     for t in range(1, NUM_TYPES))

    init = tuple(jnp.zeros((16,), jnp.int32) for _ in range(NUM_TYPES - 1))
    accs = lax.fori_loop(0, TV_REG, count_body, init)
    bounds = ([jnp.int32(0)] + [jnp.sum(a) for a in accs] + [jnp.int32(RW)])

    zero16 = jnp.zeros((16,), jnp.float32)
    for t in range(NUM_TYPES):
        for j in range(D // 16):
            s1_v[t, pl.ds(j * 16, 16)] = zero16
            s2_v[t, pl.ds(j * 16, 16)] = zero16

    def start_copy(k, buf, sem):
        off = pl.multiple_of(wstart + k * CHUNK, 8)
        pltpu.make_async_copy(
            x_hbm.at[pl.ds(off, CHUNK), :], buf, sem).start()

    def wait_copy(buf, sem):
        pltpu.make_async_copy(
            x_hbm.at[pl.ds(wstart, CHUNK), :], buf, sem).wait()

    def process(k, buf):
        base = k * CHUNK
        for t in range(NUM_TYPES):
            lo = jnp.clip(bounds[t] - base, 0, CHUNK)
            hi = jnp.clip(bounds[t + 1] - base, lo, CHUNK)

            def row_body(r, carry):
                c1, c2 = carry
                n1 = []
                n2 = []
                for j in range(D // 16):
                    v = buf[r, pl.ds(j * 16, 16)]
                    n1.append(c1[j] + v)
                    n2.append(c2[j] + v * v)
                return (tuple(n1), tuple(n2))

            rinit = (tuple(zero16 for _ in range(D // 16)),
                     tuple(zero16 for _ in range(D // 16)))
            r1, r2 = lax.fori_loop(lo, hi, row_body, rinit)

            @pl.when(hi > lo)
            def _flush():
                for j in range(D // 16):
                    s1_v[t, pl.ds(j * 16, 16)] += r1[j]
                    s2_v[t, pl.ds(j * 16, 16)] += r2[j]

    start_copy(0, xbuf.at[0], sem0)
    start_copy(1, xbuf.at[1], sem1)

    def pair_body(k, _):
        wait_copy(xbuf.at[0], sem0)
        process(2 * k, xbuf.at[0])

        @pl.when(k < NCHUNK // 2 - 1)
        def _next0():
            start_copy(2 * k + 2, xbuf.at[0], sem0)

        wait_copy(xbuf.at[1], sem1)
        process(2 * k + 1, xbuf.at[1])

        @pl.when(k < NCHUNK // 2 - 1)
        def _next1():
            start_copy(2 * k + 3, xbuf.at[1], sem1)
        return 0

    lax.fori_loop(0, NCHUNK // 2, pair_body, 0)

    for t in range(NUM_TYPES):
        c = (bounds[t + 1] - bounds[t]).astype(jnp.float32)
        cnt_v[t, :] = jnp.full((16,), c, jnp.float32)
    pltpu.sync_copy(s1_v, s1_hbm.at[wid])
    pltpu.sync_copy(s2_v, s2_hbm.at[wid])
    pltpu.sync_copy(cnt_v, cnt_hbm.at[wid])


_sc_stats = functools.partial(
    pl.kernel,
    out_type=[
        jax.ShapeDtypeStruct((NW, NUM_TYPES, D), jnp.float32),
        jax.ShapeDtypeStruct((NW, NUM_TYPES, D), jnp.float32),
        jax.ShapeDtypeStruct((NW, NUM_TYPES, 16), jnp.float32),
    ],
    mesh=plsc.VectorSubcoreMesh(core_axis_name="c", subcore_axis_name="s"),
    compiler_params=pltpu.CompilerParams(needs_layout_passes=False),
    scratch_types=[
        pltpu.VMEM((RW,), jnp.int32),
        pltpu.VMEM((2, CHUNK, D), jnp.float32),
        pltpu.VMEM((NUM_TYPES, D), jnp.float32),
        pltpu.VMEM((NUM_TYPES, D), jnp.float32),
        pltpu.VMEM((NUM_TYPES, 16), jnp.float32),
        pltpu.SemaphoreType.DMA,
        pltpu.SemaphoreType.DMA,
    ],
)(_sc_stats_body)


def _tc_stats_body(type_ref, x_ref, s1_out, s2_out, cnt_out,
                   s1_ref, s2_ref, cnt_ref):
    j = pl.program_id(0)

    @pl.when(j == 0)
    def _init():
        s1_ref[...] = jnp.zeros_like(s1_ref)
        s2_ref[...] = jnp.zeros_like(s2_ref)
        cnt_ref[...] = jnp.zeros_like(cnt_ref)

    t = type_ref[0, 0, :]
    xb = x_ref[...]
    oh = (jax.lax.broadcasted_iota(jnp.int32, (NUM_TYPES, BLK), 0)
          == t[None, :]).astype(jnp.float32)
    dn = (((1,), (0,)), ((), ()))
    s1_ref[...] += lax.dot_general(oh, xb, dn,
                                   preferred_element_type=jnp.float32)
    s2_ref[...] += lax.dot_general(oh, xb * xb, dn,
                                   preferred_element_type=jnp.float32)
    cnt_ref[...] += jnp.broadcast_to(jnp.sum(oh, axis=1)[:, None],
                                     (NUM_TYPES, D))

    @pl.when(j == NB_TC - 1)
    def _emit():
        s1_out[...] = s1_ref[...]
        s2_out[...] = s2_ref[...]
        cnt_out[...] = cnt_ref[...]


def _norm_body(type_ref, x_ref, s1_ref, s2_ref, cnt_ref,
               s1t_ref, s2t_ref, cntt_ref, out_ref,
               mean_ref, rstd_ref):
    j = pl.program_id(0)

    @pl.when(j == 0)
    def _finalize():
        c = jnp.sum(cnt_ref[...], axis=0)[:, :1] + cntt_ref[:, :1]
        c = jnp.maximum(c, 1.0)
        mean = (jnp.sum(s1_ref[...], axis=0) + s1t_ref[...]) / c
        var = (jnp.sum(s2_ref[...], axis=0) + s2t_ref[...]) / c - mean * mean
        var = jnp.maximum(var, 0.0)
        mean_ref[...] = mean
        rstd_ref[...] = lax.rsqrt(var + EPS)

    t = type_ref[0, 0, :]
    oh = (t[:, None] == lax.broadcasted_iota(
        jnp.int32, (BLK, NUM_TYPES), 1)).astype(jnp.float32)
    dn = (((1,), (0,)), ((), ()))
    m = lax.dot_general(oh, mean_ref[...], dn,
                        preferred_element_type=jnp.float32)
    r = lax.dot_general(oh, rstd_ref[...], dn,
                        preferred_element_type=jnp.float32)
    out_ref[...] = (x_ref[...] - m) * r


@jax.jit
def kernel(x, type_vec):
    tv = type_vec.astype(jnp.int32)
    tv3 = tv.reshape(NB, 1, BLK)

    s1p, s2p, cntp = _sc_stats(x, tv)

    s1t, s2t, cntt = pl.pallas_call(
        _tc_stats_body,
        grid=(NB_TC,),
        in_specs=[
            pl.BlockSpec((1, 1, BLK), lambda j: (TC_OFF + j, 0, 0)),
            pl.BlockSpec((BLK, D), lambda j: (TC_OFF + j, 0)),
        ],
        out_specs=[
            pl.BlockSpec((NUM_TYPES, D), lambda j: (0, 0)),
            pl.BlockSpec((NUM_TYPES, D), lambda j: (0, 0)),
            pl.BlockSpec((NUM_TYPES, D), lambda j: (0, 0)),
        ],
        out_shape=[
            jax.ShapeDtypeStruct((NUM_TYPES, D), jnp.float32),
            jax.ShapeDtypeStruct((NUM_TYPES, D), jnp.float32),
            jax.ShapeDtypeStruct((NUM_TYPES, D), jnp.float32),
        ],
        scratch_shapes=[
            pltpu.VMEM((NUM_TYPES, D), jnp.float32),
            pltpu.VMEM((NUM_TYPES, D), jnp.float32),
            pltpu.VMEM((NUM_TYPES, D), jnp.float32),
        ],
    )(tv3, x)

    out = pl.pallas_call(
        _norm_body,
        grid=(NB,),
        in_specs=[
            pl.BlockSpec((1, 1, BLK), lambda j: (j, 0, 0)),
            pl.BlockSpec((BLK, D), lambda j: (j, 0)),
            pl.BlockSpec((NW, NUM_TYPES, D), lambda j: (0, 0, 0)),
            pl.BlockSpec((NW, NUM_TYPES, D), lambda j: (0, 0, 0)),
            pl.BlockSpec((NW, NUM_TYPES, 16), lambda j: (0, 0, 0)),
            pl.BlockSpec((NUM_TYPES, D), lambda j: (0, 0)),
            pl.BlockSpec((NUM_TYPES, D), lambda j: (0, 0)),
            pl.BlockSpec((NUM_TYPES, D), lambda j: (0, 0)),
        ],
        out_specs=pl.BlockSpec((BLK, D), lambda j: (j, 0)),
        out_shape=jax.ShapeDtypeStruct((N, D), jnp.float32),
        scratch_shapes=[
            pltpu.VMEM((NUM_TYPES, D), jnp.float32),
            pltpu.VMEM((NUM_TYPES, D), jnp.float32),
        ],
    )(tv3, x, s1p, s2p, cntp, s1t, s2t, cntt)
    return out


# speedup vs baseline: 1.2203x; 1.1863x over previous
import functools

import jax
import jax.numpy as jnp
from jax import lax
from jax.experimental import pallas as pl
from jax.experimental.pallas import tpu as pltpu
from jax.experimental.pallas import tpu_sc as plsc

N = 320000
D = 128
NUM_TYPES = 8
EPS = 1e-05

BLK = 3200
NB = N // BLK
BLK2 = 8000
NB2 = N // BLK2

NC = 2
NS = 16
NW = NC * NS
M_SC = 179200
RW = M_SC // NW
CHUNK = 200
NCHUNK = RW // CHUNK
TV_REG = RW // 16
NB_TC = (N - M_SC) // BLK
TC_OFF = M_SC // BLK


def _sc_stats_body(x_hbm, tv_hbm, s1_hbm, s2_hbm, cnt_hbm,
                   tv_v, xbuf, s1_v, s2_v, cnt_v, sem0, sem1):
    wid = lax.axis_index("s") * NC + lax.axis_index("c")
    wstart = pl.multiple_of(wid * RW, 8)

    pltpu.sync_copy(tv_hbm.at[pl.ds(wstart, RW)], tv_v)

    def count_body(i, carry):
        tv = tv_v[pl.ds(i * 16, 16)]
        return tuple(carry[t - 1] + jnp.where(tv < t, 1, 0)
                     for t in range(1, NUM_TYPES))

    init = tuple(jnp.zeros((16,), jnp.int32) for _ in range(NUM_TYPES - 1))
    accs = lax.fori_loop(0, TV_REG, count_body, init)
    bounds = ([jnp.int32(0)] + [jnp.sum(a) for a in accs] + [jnp.int32(RW)])

    zero16 = jnp.zeros((16,), jnp.float32)
    for t in range(NUM_TYPES):
        for j in range(D // 16):
            s1_v[t, pl.ds(j * 16, 16)] = zero16
            s2_v[t, pl.ds(j * 16, 16)] = zero16

    def start_copy(k, buf, sem):
        off = pl.multiple_of(wstart + k * CHUNK, 8)
        pltpu.make_async_copy(
            x_hbm.at[pl.ds(off, CHUNK), :], buf, sem).start()

    def wait_copy(buf, sem):
        pltpu.make_async_copy(
            x_hbm.at[pl.ds(wstart, CHUNK), :], buf, sem).wait()

    def process(k, buf):
        base = k * CHUNK
        for t in range(NUM_TYPES):
            lo = jnp.clip(bounds[t] - base, 0, CHUNK)
            hi = jnp.clip(bounds[t + 1] - base, lo, CHUNK)

            def row_body(r, carry):
                c1, c2 = carry
                n1 = []
                n2 = []
                for j in range(D // 16):
                    v = buf[r, pl.ds(j * 16, 16)]
                    n1.append(c1[j] + v)
                    n2.append(c2[j] + v * v)
                return (tuple(n1), tuple(n2))

            rinit = (tuple(zero16 for _ in range(D // 16)),
                     tuple(zero16 for _ in range(D // 16)))
            r1, r2 = lax.fori_loop(lo, hi, row_body, rinit)

            @pl.when(hi > lo)
            def _flush():
                for j in range(D // 16):
                    s1_v[t, pl.ds(j * 16, 16)] += r1[j]
                    s2_v[t, pl.ds(j * 16, 16)] += r2[j]

    start_copy(0, xbuf.at[0], sem0)
    start_copy(1, xbuf.at[1], sem1)

    def pair_body(k, _):
        wait_copy(xbuf.at[0], sem0)
        process(2 * k, xbuf.at[0])

        @pl.when(k < NCHUNK // 2 - 1)
        def _next0():
            start_copy(2 * k + 2, xbuf.at[0], sem0)

        wait_copy(xbuf.at[1], sem1)
        process(2 * k + 1, xbuf.at[1])

        @pl.when(k < NCHUNK // 2 - 1)
        def _next1():
            start_copy(2 * k + 3, xbuf.at[1], sem1)
        return 0

    lax.fori_loop(0, NCHUNK // 2, pair_body, 0)

    for t in range(NUM_TYPES):
        c = (bounds[t + 1] - bounds[t]).astype(jnp.float32)
        cnt_v[t, :] = jnp.full((16,), c, jnp.float32)
    pltpu.sync_copy(s1_v, s1_hbm.at[wid])
    pltpu.sync_copy(s2_v, s2_hbm.at[wid])
    pltpu.sync_copy(cnt_v, cnt_hbm.at[wid])


_sc_stats = functools.partial(
    pl.kernel,
    out_type=[
        jax.ShapeDtypeStruct((NW, NUM_TYPES, D), jnp.float32),
        jax.ShapeDtypeStruct((NW, NUM_TYPES, D), jnp.float32),
        jax.ShapeDtypeStruct((NW, NUM_TYPES, 16), jnp.float32),
    ],
    mesh=plsc.VectorSubcoreMesh(core_axis_name="c", subcore_axis_name="s"),
    compiler_params=pltpu.CompilerParams(needs_layout_passes=False),
    scratch_types=[
        pltpu.VMEM((RW,), jnp.int32),
        pltpu.VMEM((2, CHUNK, D), jnp.float32),
        pltpu.VMEM((NUM_TYPES, D), jnp.float32),
        pltpu.VMEM((NUM_TYPES, D), jnp.float32),
        pltpu.VMEM((NUM_TYPES, 16), jnp.float32),
        pltpu.SemaphoreType.DMA,
        pltpu.SemaphoreType.DMA,
    ],
)(_sc_stats_body)


def _tc_stats_body(type_ref, x_ref, s1_out, s2_out, cnt_out,
                   s1_ref, s2_ref, cnt_ref):
    j = pl.program_id(0)

    @pl.when(j == 0)
    def _init():
        s1_ref[...] = jnp.zeros_like(s1_ref)
        s2_ref[...] = jnp.zeros_like(s2_ref)
        cnt_ref[...] = jnp.zeros_like(cnt_ref)

    t = type_ref[0, 0, :]
    xb = x_ref[...]
    oh = (jax.lax.broadcasted_iota(jnp.int32, (NUM_TYPES, BLK), 0)
          == t[None, :]).astype(jnp.float32)
    dn = (((1,), (0,)), ((), ()))
    s1_ref[...] += lax.dot_general(oh, xb, dn,
                                   preferred_element_type=jnp.float32)
    s2_ref[...] += lax.dot_general(oh, xb * xb, dn,
                                   preferred_element_type=jnp.float32)
    cnt_ref[...] += jnp.broadcast_to(jnp.sum(oh, axis=1)[:, None],
                                     (NUM_TYPES, D))

    @pl.when(j == NB_TC - 1)
    def _emit():
        s1_out[...] = s1_ref[...]
        s2_out[...] = s2_ref[...]
        cnt_out[...] = cnt_ref[...]


def _norm_body(type_ref, x_ref, s1_ref, s2_ref, cnt_ref,
               s1t_ref, s2t_ref, cntt_ref, out_ref,
               mean_ref, rstd_ref):
    j = pl.program_id(0)

    @pl.when(j == 0)
    def _finalize():
        c = jnp.sum(cnt_ref[...], axis=0)[:, :1] + cntt_ref[:, :1]
        c = jnp.maximum(c, 1.0)
        mean = (jnp.sum(s1_ref[...], axis=0) + s1t_ref[...]) / c
        var = (jnp.sum(s2_ref[...], axis=0) + s2t_ref[...]) / c - mean * mean
        var = jnp.maximum(var, 0.0)
        mean_ref[...] = mean
        rstd_ref[...] = lax.rsqrt(var + EPS)

    t = type_ref[0, 0, :]
    oh = (t[:, None] == lax.broadcasted_iota(
        jnp.int32, (BLK2, NUM_TYPES), 1)).astype(jnp.float32)
    dn = (((1,), (0,)), ((), ()))
    m = lax.dot_general(oh, mean_ref[...], dn,
                        preferred_element_type=jnp.float32)
    r = lax.dot_general(oh, rstd_ref[...], dn,
                        preferred_element_type=jnp.float32)
    out_ref[...] = (x_ref[...] - m) * r


@jax.jit
def kernel(x, type_vec):
    tv = type_vec.astype(jnp.int32)
    tv3 = tv.reshape(NB, 1, BLK)

    s1p, s2p, cntp = _sc_stats(x, tv)

    s1t, s2t, cntt = pl.pallas_call(
        _tc_stats_body,
        grid=(NB_TC,),
        in_specs=[
            pl.BlockSpec((1, 1, BLK), lambda j: (TC_OFF + j, 0, 0)),
            pl.BlockSpec((BLK, D), lambda j: (TC_OFF + j, 0)),
        ],
        out_specs=[
            pl.BlockSpec((NUM_TYPES, D), lambda j: (0, 0)),
            pl.BlockSpec((NUM_TYPES, D), lambda j: (0, 0)),
            pl.BlockSpec((NUM_TYPES, D), lambda j: (0, 0)),
        ],
        out_shape=[
            jax.ShapeDtypeStruct((NUM_TYPES, D), jnp.float32),
            jax.ShapeDtypeStruct((NUM_TYPES, D), jnp.float32),
            jax.ShapeDtypeStruct((NUM_TYPES, D), jnp.float32),
        ],
        scratch_shapes=[
            pltpu.VMEM((NUM_TYPES, D), jnp.float32),
            pltpu.VMEM((NUM_TYPES, D), jnp.float32),
            pltpu.VMEM((NUM_TYPES, D), jnp.float32),
        ],
    )(tv3, x)

    tv3n = tv.reshape(NB2, 1, BLK2)
    out = pl.pallas_call(
        _norm_body,
        grid=(NB2,),
        in_specs=[
            pl.BlockSpec((1, 1, BLK2), lambda j: (j, 0, 0)),
            pl.BlockSpec((BLK2, D), lambda j: (j, 0)),
            pl.BlockSpec((NW, NUM_TYPES, D), lambda j: (0, 0, 0)),
            pl.BlockSpec((NW, NUM_TYPES, D), lambda j: (0, 0, 0)),
            pl.BlockSpec((NW, NUM_TYPES, 16), lambda j: (0, 0, 0)),
            pl.BlockSpec((NUM_TYPES, D), lambda j: (0, 0)),
            pl.BlockSpec((NUM_TYPES, D), lambda j: (0, 0)),
            pl.BlockSpec((NUM_TYPES, D), lambda j: (0, 0)),
        ],
        out_specs=pl.BlockSpec((BLK2, D), lambda j: (j, 0)),
        out_shape=jax.ShapeDtypeStruct((N, D), jnp.float32),
        scratch_shapes=[
            pltpu.VMEM((NUM_TYPES, D), jnp.float32),
            pltpu.VMEM((NUM_TYPES, D), jnp.float32),
        ],
    )(tv3n, x, s1p, s2p, cntp, s1t, s2t, cntt)
    return out


# speedup vs baseline: 1.2492x; 1.0237x over previous
import functools

import jax
import jax.numpy as jnp
from jax import lax
from jax.experimental import pallas as pl
from jax.experimental.pallas import tpu as pltpu
from jax.experimental.pallas import tpu_sc as plsc

N = 320000
D = 128
NUM_TYPES = 8
EPS = 1e-05

BLK = 3200
NB = N // BLK
BLK2 = 16000
NB2 = N // BLK2

NC = 2
NS = 16
NW = NC * NS
M_SC = 179200
RW = M_SC // NW
CHUNK = 200
NCHUNK = RW // CHUNK
TV_REG = RW // 16
NB_TC = (N - M_SC) // BLK
TC_OFF = M_SC // BLK


def _sc_stats_body(x_hbm, tv_hbm, s1_hbm, s2_hbm, cnt_hbm,
                   tv_v, xbuf, s1_v, s2_v, cnt_v, sem0, sem1):
    wid = lax.axis_index("s") * NC + lax.axis_index("c")
    wstart = pl.multiple_of(wid * RW, 8)

    pltpu.sync_copy(tv_hbm.at[pl.ds(wstart, RW)], tv_v)

    def count_body(i, carry):
        tv = tv_v[pl.ds(i * 16, 16)]
        return tuple(carry[t - 1] + jnp.where(tv < t, 1, 0)
                     for t in range(1, NUM_TYPES))

    init = tuple(jnp.zeros((16,), jnp.int32) for _ in range(NUM_TYPES - 1))
    accs = lax.fori_loop(0, TV_REG, count_body, init)
    bounds = ([jnp.int32(0)] + [jnp.sum(a) for a in accs] + [jnp.int32(RW)])

    zero16 = jnp.zeros((16,), jnp.float32)
    for t in range(NUM_TYPES):
        for j in range(D // 16):
            s1_v[t, pl.ds(j * 16, 16)] = zero16
            s2_v[t, pl.ds(j * 16, 16)] = zero16

    def start_copy(k, buf, sem):
        off = pl.multiple_of(wstart + k * CHUNK, 8)
        pltpu.make_async_copy(
            x_hbm.at[pl.ds(off, CHUNK), :], buf, sem).start()

    def wait_copy(buf, sem):
        pltpu.make_async_copy(
            x_hbm.at[pl.ds(wstart, CHUNK), :], buf, sem).wait()

    def process(k, buf):
        base = k * CHUNK
        for t in range(NUM_TYPES):
            lo = jnp.clip(bounds[t] - base, 0, CHUNK)
            hi = jnp.clip(bounds[t + 1] - base, lo, CHUNK)

            def row_body(r, carry):
                c1, c2 = carry
                n1 = []
                n2 = []
                for j in range(D // 16):
                    v = buf[r, pl.ds(j * 16, 16)]
                    n1.append(c1[j] + v)
                    n2.append(c2[j] + v * v)
                return (tuple(n1), tuple(n2))

            rinit = (tuple(zero16 for _ in range(D // 16)),
                     tuple(zero16 for _ in range(D // 16)))
            r1, r2 = lax.fori_loop(lo, hi, row_body, rinit)

            @pl.when(hi > lo)
            def _flush():
                for j in range(D // 16):
                    s1_v[t, pl.ds(j * 16, 16)] += r1[j]
                    s2_v[t, pl.ds(j * 16, 16)] += r2[j]

    start_copy(0, xbuf.at[0], sem0)
    start_copy(1, xbuf.at[1], sem1)

    def pair_body(k, _):
        wait_copy(xbuf.at[0], sem0)
        process(2 * k, xbuf.at[0])

        @pl.when(k < NCHUNK // 2 - 1)
        def _next0():
            start_copy(2 * k + 2, xbuf.at[0], sem0)

        wait_copy(xbuf.at[1], sem1)
        process(2 * k + 1, xbuf.at[1])

        @pl.when(k < NCHUNK // 2 - 1)
        def _next1():
            start_copy(2 * k + 3, xbuf.at[1], sem1)
        return 0

    lax.fori_loop(0, NCHUNK // 2, pair_body, 0)

    for t in range(NUM_TYPES):
        c = (bounds[t + 1] - bounds[t]).astype(jnp.float32)
        cnt_v[t, :] = jnp.full((16,), c, jnp.float32)
    pltpu.sync_copy(s1_v, s1_hbm.at[wid])
    pltpu.sync_copy(s2_v, s2_hbm.at[wid])
    pltpu.sync_copy(cnt_v, cnt_hbm.at[wid])


_sc_stats = functools.partial(
    pl.kernel,
    out_type=[
        jax.ShapeDtypeStruct((NW, NUM_TYPES, D), jnp.float32),
        jax.ShapeDtypeStruct((NW, NUM_TYPES, D), jnp.float32),
        jax.ShapeDtypeStruct((NW, NUM_TYPES, 16), jnp.float32),
    ],
    mesh=plsc.VectorSubcoreMesh(core_axis_name="c", subcore_axis_name="s"),
    compiler_params=pltpu.CompilerParams(needs_layout_passes=False),
    scratch_types=[
        pltpu.VMEM((RW,), jnp.int32),
        pltpu.VMEM((2, CHUNK, D), jnp.float32),
        pltpu.VMEM((NUM_TYPES, D), jnp.float32),
        pltpu.VMEM((NUM_TYPES, D), jnp.float32),
        pltpu.VMEM((NUM_TYPES, 16), jnp.float32),
        pltpu.SemaphoreType.DMA,
        pltpu.SemaphoreType.DMA,
    ],
)(_sc_stats_body)


def _tc_stats_body(type_ref, x_ref, s1_out, s2_out, cnt_out,
                   s1_ref, s2_ref, cnt_ref):
    j = pl.program_id(0)

    @pl.when(j == 0)
    def _init():
        s1_ref[...] = jnp.zeros_like(s1_ref)
        s2_ref[...] = jnp.zeros_like(s2_ref)
        cnt_ref[...] = jnp.zeros_like(cnt_ref)

    t = type_ref[0, 0, :]
    xb = x_ref[...]
    oh = (jax.lax.broadcasted_iota(jnp.int32, (NUM_TYPES, BLK), 0)
          == t[None, :]).astype(jnp.float32)
    dn = (((1,), (0,)), ((), ()))
    s1_ref[...] += lax.dot_general(oh, xb, dn,
                                   preferred_element_type=jnp.float32)
    s2_ref[...] += lax.dot_general(oh, xb * xb, dn,
                                   preferred_element_type=jnp.float32)
    cnt_ref[...] += jnp.broadcast_to(jnp.sum(oh, axis=1)[:, None],
                                     (NUM_TYPES, D))

    @pl.when(j == NB_TC - 1)
    def _emit():
        s1_out[...] = s1_ref[...]
        s2_out[...] = s2_ref[...]
        cnt_out[...] = cnt_ref[...]


def _norm_body(type_ref, x_ref, s1_ref, s2_ref, cnt_ref,
               s1t_ref, s2t_ref, cntt_ref, out_ref,
               mean_ref, rstd_ref):
    j = pl.program_id(0)

    @pl.when(j == 0)
    def _finalize():
        c = jnp.sum(cnt_ref[...], axis=0)[:, :1] + cntt_ref[:, :1]
        c = jnp.maximum(c, 1.0)
        mean = (jnp.sum(s1_ref[...], axis=0) + s1t_ref[...]) / c
        var = (jnp.sum(s2_ref[...], axis=0) + s2t_ref[...]) / c - mean * mean
        var = jnp.maximum(var, 0.0)
        mean_ref[...] = mean
        rstd_ref[...] = lax.rsqrt(var + EPS)

    t = type_ref[0, 0, :]
    oh = (t[:, None] == lax.broadcasted_iota(
        jnp.int32, (BLK2, NUM_TYPES), 1)).astype(jnp.float32)
    dn = (((1,), (0,)), ((), ()))
    m = lax.dot_general(oh, mean_ref[...], dn,
                        preferred_element_type=jnp.float32)
    r = lax.dot_general(oh, rstd_ref[...], dn,
                        preferred_element_type=jnp.float32)
    out_ref[...] = (x_ref[...] - m) * r


@jax.jit
def kernel(x, type_vec):
    tv = type_vec.astype(jnp.int32)
    tv3 = tv.reshape(NB, 1, BLK)

    s1p, s2p, cntp = _sc_stats(x, tv)

    s1t, s2t, cntt = pl.pallas_call(
        _tc_stats_body,
        grid=(NB_TC,),
        in_specs=[
            pl.BlockSpec((1, 1, BLK), lambda j: (TC_OFF + j, 0, 0)),
            pl.BlockSpec((BLK, D), lambda j: (TC_OFF + j, 0)),
        ],
        out_specs=[
            pl.BlockSpec((NUM_TYPES, D), lambda j: (0, 0)),
            pl.BlockSpec((NUM_TYPES, D), lambda j: (0, 0)),
            pl.BlockSpec((NUM_TYPES, D), lambda j: (0, 0)),
        ],
        out_shape=[
            jax.ShapeDtypeStruct((NUM_TYPES, D), jnp.float32),
            jax.ShapeDtypeStruct((NUM_TYPES, D), jnp.float32),
            jax.ShapeDtypeStruct((NUM_TYPES, D), jnp.float32),
        ],
        scratch_shapes=[
            pltpu.VMEM((NUM_TYPES, D), jnp.float32),
            pltpu.VMEM((NUM_TYPES, D), jnp.float32),
            pltpu.VMEM((NUM_TYPES, D), jnp.float32),
        ],
    )(tv3, x)

    tv3n = tv.reshape(NB2, 1, BLK2)
    out = pl.pallas_call(
        _norm_body,
        grid=(NB2,),
        in_specs=[
            pl.BlockSpec((1, 1, BLK2), lambda j: (j, 0, 0)),
            pl.BlockSpec((BLK2, D), lambda j: (j, 0)),
            pl.BlockSpec((NW, NUM_TYPES, D), lambda j: (0, 0, 0)),
            pl.BlockSpec((NW, NUM_TYPES, D), lambda j: (0, 0, 0)),
            pl.BlockSpec((NW, NUM_TYPES, 16), lambda j: (0, 0, 0)),
            pl.BlockSpec((NUM_TYPES, D), lambda j: (0, 0)),
            pl.BlockSpec((NUM_TYPES, D), lambda j: (0, 0)),
            pl.BlockSpec((NUM_TYPES, D), lambda j: (0, 0)),
        ],
        out_specs=pl.BlockSpec((BLK2, D), lambda j: (j, 0)),
        out_shape=jax.ShapeDtypeStruct((N, D), jnp.float32),
        scratch_shapes=[
            pltpu.VMEM((NUM_TYPES, D), jnp.float32),
            pltpu.VMEM((NUM_TYPES, D), jnp.float32),
        ],
    )(tv3n, x, s1p, s2p, cntp, s1t, s2t, cntt)
    return out


# speedup vs baseline: 1.2498x; 1.0005x over previous
import functools

import jax
import jax.numpy as jnp
from jax import lax
from jax.experimental import pallas as pl
from jax.experimental.pallas import tpu as pltpu
from jax.experimental.pallas import tpu_sc as plsc

N = 320000
D = 128
NUM_TYPES = 8
EPS = 1e-05

BLK = 3200
NB = N // BLK
BLK2 = 20000
NB2 = N // BLK2

NC = 2
NS = 16
NW = NC * NS
M_SC = 179200
RW = M_SC // NW
CHUNK = 200
NCHUNK = RW // CHUNK
TV_REG = RW // 16
NB_TC = (N - M_SC) // BLK
TC_OFF = M_SC // BLK


def _sc_stats_body(x_hbm, tv_hbm, s1_hbm, s2_hbm, cnt_hbm,
                   tv_v, xbuf, s1_v, s2_v, cnt_v, sem0, sem1):
    wid = lax.axis_index("s") * NC + lax.axis_index("c")
    wstart = pl.multiple_of(wid * RW, 8)

    pltpu.sync_copy(tv_hbm.at[pl.ds(wstart, RW)], tv_v)

    def count_body(i, carry):
        tv = tv_v[pl.ds(i * 16, 16)]
        return tuple(carry[t - 1] + jnp.where(tv < t, 1, 0)
                     for t in range(1, NUM_TYPES))

    init = tuple(jnp.zeros((16,), jnp.int32) for _ in range(NUM_TYPES - 1))
    accs = lax.fori_loop(0, TV_REG, count_body, init)
    bounds = ([jnp.int32(0)] + [jnp.sum(a) for a in accs] + [jnp.int32(RW)])

    zero16 = jnp.zeros((16,), jnp.float32)
    for t in range(NUM_TYPES):
        for j in range(D // 16):
            s1_v[t, pl.ds(j * 16, 16)] = zero16
            s2_v[t, pl.ds(j * 16, 16)] = zero16

    def start_copy(k, buf, sem):
        off = pl.multiple_of(wstart + k * CHUNK, 8)
        pltpu.make_async_copy(
            x_hbm.at[pl.ds(off, CHUNK), :], buf, sem).start()

    def wait_copy(buf, sem):
        pltpu.make_async_copy(
            x_hbm.at[pl.ds(wstart, CHUNK), :], buf, sem).wait()

    def process(k, buf):
        base = k * CHUNK
        for t in range(NUM_TYPES):
            lo = jnp.clip(bounds[t] - base, 0, CHUNK)
            hi = jnp.clip(bounds[t + 1] - base, lo, CHUNK)

            def row_body(r, carry):
                c1, c2 = carry
                n1 = []
                n2 = []
                for j in range(D // 16):
                    v = buf[r, pl.ds(j * 16, 16)]
                    n1.append(c1[j] + v)
                    n2.append(c2[j] + v * v)
                return (tuple(n1), tuple(n2))

            rinit = (tuple(zero16 for _ in range(D // 16)),
                     tuple(zero16 for _ in range(D // 16)))
            r1, r2 = lax.fori_loop(lo, hi, row_body, rinit)

            @pl.when(hi > lo)
            def _flush():
                for j in range(D // 16):
                    s1_v[t, pl.ds(j * 16, 16)] += r1[j]
                    s2_v[t, pl.ds(j * 16, 16)] += r2[j]

    start_copy(0, xbuf.at[0], sem0)
    start_copy(1, xbuf.at[1], sem1)

    def pair_body(k, _):
        wait_copy(xbuf.at[0], sem0)
        process(2 * k, xbuf.at[0])

        @pl.when(k < NCHUNK // 2 - 1)
        def _next0():
            start_copy(2 * k + 2, xbuf.at[0], sem0)

        wait_copy(xbuf.at[1], sem1)
        process(2 * k + 1, xbuf.at[1])

        @pl.when(k < NCHUNK // 2 - 1)
        def _next1():
            start_copy(2 * k + 3, xbuf.at[1], sem1)
        return 0

    lax.fori_loop(0, NCHUNK // 2, pair_body, 0)

    for t in range(NUM_TYPES):
        c = (bounds[t + 1] - bounds[t]).astype(jnp.float32)
        cnt_v[t, :] = jnp.full((16,), c, jnp.float32)
    pltpu.sync_copy(s1_v, s1_hbm.at[wid])
    pltpu.sync_copy(s2_v, s2_hbm.at[wid])
    pltpu.sync_copy(cnt_v, cnt_hbm.at[wid])


_sc_stats = functools.partial(
    pl.kernel,
    out_type=[
        jax.ShapeDtypeStruct((NW, NUM_TYPES, D), jnp.float32),
        jax.ShapeDtypeStruct((NW, NUM_TYPES, D), jnp.float32),
        jax.ShapeDtypeStruct((NW, NUM_TYPES, 16), jnp.float32),
    ],
    mesh=plsc.VectorSubcoreMesh(core_axis_name="c", subcore_axis_name="s"),
    compiler_params=pltpu.CompilerParams(needs_layout_passes=False),
    scratch_types=[
        pltpu.VMEM((RW,), jnp.int32),
        pltpu.VMEM((2, CHUNK, D), jnp.float32),
        pltpu.VMEM((NUM_TYPES, D), jnp.float32),
        pltpu.VMEM((NUM_TYPES, D), jnp.float32),
        pltpu.VMEM((NUM_TYPES, 16), jnp.float32),
        pltpu.SemaphoreType.DMA,
        pltpu.SemaphoreType.DMA,
    ],
)(_sc_stats_body)


def _tc_stats_body(type_ref, x_ref, s1_out, s2_out, cnt_out,
                   s1_ref, s2_ref, cnt_ref):
    j = pl.program_id(0)

    @pl.when(j == 0)
    def _init():
        s1_ref[...] = jnp.zeros_like(s1_ref)
        s2_ref[...] = jnp.zeros_like(s2_ref)
        cnt_ref[...] = jnp.zeros_like(cnt_ref)

    t = type_ref[0, 0, :]
    xb = x_ref[...]
    oh = (jax.lax.broadcasted_iota(jnp.int32, (NUM_TYPES, BLK), 0)
          == t[None, :]).astype(jnp.float32)
    dn = (((1,), (0,)), ((), ()))
    s1_ref[...] += lax.dot_general(oh, xb, dn,
                                   preferred_element_type=jnp.float32)
    s2_ref[...] += lax.dot_general(oh, xb * xb, dn,
                                   preferred_element_type=jnp.float32)
    cnt_ref[...] += jnp.broadcast_to(jnp.sum(oh, axis=1)[:, None],
                                     (NUM_TYPES, D))

    @pl.when(j == NB_TC - 1)
    def _emit():
        s1_out[...] = s1_ref[...]
        s2_out[...] = s2_ref[...]
        cnt_out[...] = cnt_ref[...]


def _norm_body(type_ref, x_ref, s1_ref, s2_ref, cnt_ref,
               s1t_ref, s2t_ref, cntt_ref, out_ref,
               mean_ref, rstd_ref):
    j = pl.program_id(0)

    @pl.when(j == 0)
    def _finalize():
        c = jnp.sum(cnt_ref[...], axis=0)[:, :1] + cntt_ref[:, :1]
        c = jnp.maximum(c, 1.0)
        mean = (jnp.sum(s1_ref[...], axis=0) + s1t_ref[...]) / c
        var = (jnp.sum(s2_ref[...], axis=0) + s2t_ref[...]) / c - mean * mean
        var = jnp.maximum(var, 0.0)
        mean_ref[...] = mean
        rstd_ref[...] = lax.rsqrt(var + EPS)

    t = type_ref[0, 0, :]
    oh = (t[:, None] == lax.broadcasted_iota(
        jnp.int32, (BLK2, NUM_TYPES), 1)).astype(jnp.float32)
    dn = (((1,), (0,)), ((), ()))
    m = lax.dot_general(oh, mean_ref[...], dn,
                        preferred_element_type=jnp.float32)
    r = lax.dot_general(oh, rstd_ref[...], dn,
                        preferred_element_type=jnp.float32)
    out_ref[...] = (x_ref[...] - m) * r


@jax.jit
def kernel(x, type_vec):
    tv = type_vec.astype(jnp.int32)
    tv3 = tv.reshape(NB, 1, BLK)

    s1p, s2p, cntp = _sc_stats(x, tv)

    s1t, s2t, cntt = pl.pallas_call(
        _tc_stats_body,
        grid=(NB_TC,),
        in_specs=[
            pl.BlockSpec((1, 1, BLK), lambda j: (TC_OFF + j, 0, 0)),
            pl.BlockSpec((BLK, D), lambda j: (TC_OFF + j, 0)),
        ],
        out_specs=[
            pl.BlockSpec((NUM_TYPES, D), lambda j: (0, 0)),
            pl.BlockSpec((NUM_TYPES, D), lambda j: (0, 0)),
            pl.BlockSpec((NUM_TYPES, D), lambda j: (0, 0)),
        ],
        out_shape=[
            jax.ShapeDtypeStruct((NUM_TYPES, D), jnp.float32),
            jax.ShapeDtypeStruct((NUM_TYPES, D), jnp.float32),
            jax.ShapeDtypeStruct((NUM_TYPES, D), jnp.float32),
        ],
        scratch_shapes=[
            pltpu.VMEM((NUM_TYPES, D), jnp.float32),
            pltpu.VMEM((NUM_TYPES, D), jnp.float32),
            pltpu.VMEM((NUM_TYPES, D), jnp.float32),
        ],
    )(tv3, x)

    tv3n = tv.reshape(NB2, 1, BLK2)
    out = pl.pallas_call(
        _norm_body,
        grid=(NB2,),
        in_specs=[
            pl.BlockSpec((1, 1, BLK2), lambda j: (j, 0, 0)),
            pl.BlockSpec((BLK2, D), lambda j: (j, 0)),
            pl.BlockSpec((NW, NUM_TYPES, D), lambda j: (0, 0, 0)),
            pl.BlockSpec((NW, NUM_TYPES, D), lambda j: (0, 0, 0)),
            pl.BlockSpec((NW, NUM_TYPES, 16), lambda j: (0, 0, 0)),
            pl.BlockSpec((NUM_TYPES, D), lambda j: (0, 0)),
            pl.BlockSpec((NUM_TYPES, D), lambda j: (0, 0)),
            pl.BlockSpec((NUM_TYPES, D), lambda j: (0, 0)),
        ],
        out_specs=pl.BlockSpec((BLK2, D), lambda j: (j, 0)),
        out_shape=jax.ShapeDtypeStruct((N, D), jnp.float32),
        scratch_shapes=[
            pltpu.VMEM((NUM_TYPES, D), jnp.float32),
            pltpu.VMEM((NUM_TYPES, D), jnp.float32),
        ],
    )(tv3n, x, s1p, s2p, cntp, s1t, s2t, cntt)
    return out
